# baseline (speedup 1.0000x reference)
import functools

import jax
import jax.numpy as jnp
from jax import lax
from jax.experimental import pallas as pl
from jax.experimental.pallas import tpu as pltpu
from jax.experimental.pallas import tpu_sc as plsc

_N = 10000
_NP = 10240
_E = 320000
_H = 128
_NL = 3
_BS = 400
_NB = _N // _BS
_B = 40
_NSUB = 16
_EPT = _E // _NSUB
_NCH = _EPT // _B
_ROWS = _NP // _NSUB
_DEGW = 16
_F32 = jnp.float32


def _enc_body(a_ref, w_ref, b_ref, g_ref, bt_ref, o_ref):
    x = jnp.dot(a_ref[...], w_ref[...], preferred_element_type=_F32) + b_ref[...]
    x = jnp.maximum(x, 0.0)
    m = jnp.mean(x, axis=-1, keepdims=True)
    v = jnp.mean((x - m) ** 2, axis=-1, keepdims=True)
    o_ref[...] = (x - m) * lax.rsqrt(v + 1e-5) * g_ref[...] + bt_ref[...]


def _enc(atomp, wp, b, g, bt):
    return pl.pallas_call(
        _enc_body,
        grid=(_NB,),
        in_specs=[
            pl.BlockSpec((_BS, _H), lambda i: (i, 0)),
            pl.BlockSpec((_H, _H), lambda i: (0, 0)),
            pl.BlockSpec((1, _H), lambda i: (0, 0)),
            pl.BlockSpec((1, _H), lambda i: (0, 0)),
            pl.BlockSpec((1, _H), lambda i: (0, 0)),
        ],
        out_specs=pl.BlockSpec((_BS, _H), lambda i: (i, 0)),
        out_shape=jax.ShapeDtypeStruct((_N, _H), _F32),
    )(atomp, wp, b, g, bt)


def _pq_body(x_ref, w_ref, b_ref, o_ref):
    r = pl.program_id(0) * 2 + pl.program_id(1)
    rows = lax.broadcasted_iota(jnp.int32, (8, _H), 0)
    b = jnp.sum(jnp.where(rows == r, b_ref[...], 0.0), axis=0, keepdims=True)
    o_ref[...] = (
        jnp.dot(x_ref[...], w_ref[0], preferred_element_type=_F32) + b
    )


def _pq(x, wst, bst8):
    return pl.pallas_call(
        _pq_body,
        grid=(2, 2, _NB),
        in_specs=[
            pl.BlockSpec((_BS, _H), lambda t, c, i: (i, 0)),
            pl.BlockSpec((1, _H, _H), lambda t, c, i: (t, 0, c)),
            pl.BlockSpec((8, _H), lambda t, c, i: (0, 0)),
        ],
        out_specs=pl.BlockSpec((_BS, _H), lambda t, c, i: (t * 2 * _NB + c * _NB + i, 0)),
        out_shape=jax.ShapeDtypeStruct((4 * _N, _H), _F32),
    )(x, wst, bst8)


def _gru_ln(s0, s1, x, deg, w2_ref, b2_ref, wih_ref, bih_ref,
            whh_ref, bhh_ref, lg_ref, lb_ref):
    aggr = (
        jnp.dot(s0, w2_ref[0], preferred_element_type=_F32)
        + jnp.dot(s1, w2_ref[1], preferred_element_type=_F32)
        + deg * b2_ref[...]
    )
    gi = jnp.dot(aggr, wih_ref[...], preferred_element_type=_F32) + bih_ref[...]
    gh = jnp.dot(x, whh_ref[...], preferred_element_type=_F32) + bhh_ref[...]
    r = jax.nn.sigmoid(gi[:, :_H] + gh[:, :_H])
    z = jax.nn.sigmoid(gi[:, _H:2 * _H] + gh[:, _H:2 * _H])
    n = jnp.tanh(gi[:, 2 * _H:] + r * gh[:, 2 * _H:])
    h = x + (1.0 - z) * n + z * x
    m = jnp.mean(h, axis=-1, keepdims=True)
    v = jnp.mean((h - m) ** 2, axis=-1, keepdims=True)
    return (h - m) * lax.rsqrt(v + 1e-5) * lg_ref[...] + lb_ref[...]


def _tcb_body(s0_ref, s1_ref, x_ref, deg_ref, w2_ref, b2_ref, wih_ref,
              bih_ref, whh_ref, bhh_ref, lg_ref, lb_ref, o_ref):
    o_ref[...] = _gru_ln(s0_ref[...], s1_ref[...], x_ref[...], deg_ref[...],
                         w2_ref, b2_ref, wih_ref, bih_ref, whh_ref, bhh_ref,
                         lg_ref, lb_ref)


def _tcb(S2, x, deg128, w2st, b2, wih, bih, whh, bhh, lg, lb):
    return pl.pallas_call(
        _tcb_body,
        grid=(_NB,),
        in_specs=[
            pl.BlockSpec((_BS, _H), lambda i: (i, 0)),
            pl.BlockSpec((_BS, _H), lambda i: (i, 0)),
            pl.BlockSpec((_BS, _H), lambda i: (i, 0)),
            pl.BlockSpec((_BS, _H), lambda i: (i, 0)),
            pl.BlockSpec((2, _H, _H), lambda i: (0, 0, 0)),
            pl.BlockSpec((1, _H), lambda i: (0, 0)),
            pl.BlockSpec((_H, 3 * _H), lambda i: (0, 0)),
            pl.BlockSpec((1, 3 * _H), lambda i: (0, 0)),
            pl.BlockSpec((_H, 3 * _H), lambda i: (0, 0)),
            pl.BlockSpec((1, 3 * _H), lambda i: (0, 0)),
            pl.BlockSpec((1, _H), lambda i: (0, 0)),
            pl.BlockSpec((1, _H), lambda i: (0, 0)),
        ],
        out_specs=pl.BlockSpec((_BS, _H), lambda i: (i, 0)),
        out_shape=jax.ShapeDtypeStruct((_N, _H), _F32),
    )(S2[:_N], S2[_NP:_NP + _N], x, deg128, w2st, b2, wih, bih, whh, bhh,
      lg, lb)


def _tcb_final_body(s0_ref, s1_ref, x_ref, deg_ref, w2_ref, b2_ref, wih_ref,
                    bih_ref, whh_ref, bhh_ref, lg_ref, lb_ref, w1_ref, b1_ref,
                    w2r_ref, b2r_ref, w3_ref, b3_ref, o_ref, sum_acc, max_acc):
    @pl.when(pl.program_id(0) == 0)
    def _init():
        sum_acc[...] = jnp.zeros_like(sum_acc)
        max_acc[...] = jnp.full_like(max_acc, -jnp.inf)

    h = _gru_ln(s0_ref[...], s1_ref[...], x_ref[...], deg_ref[...],
                w2_ref, b2_ref, wih_ref, bih_ref, whh_ref, bhh_ref,
                lg_ref, lb_ref)
    sum_acc[...] += jnp.sum(h, axis=0, keepdims=True)
    max_acc[...] = jnp.maximum(max_acc[...], jnp.max(h, axis=0, keepdims=True))

    @pl.when(pl.program_id(0) == _NB - 1)
    def _final():
        g = jnp.concatenate([sum_acc[...] / float(_N), max_acc[...]], axis=-1)
        h1 = jnp.maximum(
            jnp.dot(g, w1_ref[...], preferred_element_type=_F32) + b1_ref[...], 0.0)
        h2 = jnp.maximum(
            jnp.dot(h1, w2r_ref[...], preferred_element_type=_F32) + b2r_ref[...], 0.0)
        o_ref[...] = jnp.dot(h2, w3_ref[...], preferred_element_type=_F32) + b3_ref[...]


def _tcb_final(S2, x, deg128, w2st, b2, wih, bih, whh, bhh, lg, lb,
               rw1, rb1, rw2, rb2, rw3, rb3):
    return pl.pallas_call(
        _tcb_final_body,
        grid=(_NB,),
        in_specs=[
            pl.BlockSpec((_BS, _H), lambda i: (i, 0)),
            pl.BlockSpec((_BS, _H), lambda i: (i, 0)),
            pl.BlockSpec((_BS, _H), lambda i: (i, 0)),
            pl.BlockSpec((_BS, _H), lambda i: (i, 0)),
            pl.BlockSpec((2, _H, _H), lambda i: (0, 0, 0)),
            pl.BlockSpec((1, _H), lambda i: (0, 0)),
            pl.BlockSpec((_H, 3 * _H), lambda i: (0, 0)),
            pl.BlockSpec((1, 3 * _H), lambda i: (0, 0)),
            pl.BlockSpec((_H, 3 * _H), lambda i: (0, 0)),
            pl.BlockSpec((1, 3 * _H), lambda i: (0, 0)),
            pl.BlockSpec((1, _H), lambda i: (0, 0)),
            pl.BlockSpec((1, _H), lambda i: (0, 0)),
            pl.BlockSpec((2 * _H, _H), lambda i: (0, 0)),
            pl.BlockSpec((1, _H), lambda i: (0, 0)),
            pl.BlockSpec((_H, _H), lambda i: (0, 0)),
            pl.BlockSpec((1, _H), lambda i: (0, 0)),
            pl.BlockSpec((_H, _H), lambda i: (0, 0)),
            pl.BlockSpec((1, _H), lambda i: (0, 0)),
        ],
        out_specs=pl.BlockSpec((1, _H), lambda i: (0, 0)),
        out_shape=jax.ShapeDtypeStruct((1, _H), _F32),
        scratch_shapes=[
            pltpu.VMEM((1, _H), _F32),
            pltpu.VMEM((1, _H), _F32),
        ],
    )(S2[:_N], S2[_NP:_NP + _N], x, deg128, w2st, b2, wih, bih, whh, bhh,
      lg, lb, rw1, rb1, rw2, rb2, rw3, rb3)


def _sc_edge_fn():
    mesh = plsc.VectorSubcoreMesh(core_axis_name="c", subcore_axis_name="s")
    scratch = [
        pltpu.VMEM((2, _B), jnp.int32),
        pltpu.VMEM((2, _B), jnp.int32),
        pltpu.VMEM((2, _B), jnp.int32),
        pltpu.VMEM((_B,), jnp.int32),
        pltpu.VMEM((_B,), jnp.int32),
        pltpu.VMEM((2, _B, 16), _F32),
        pltpu.VMEM((2, _B, _H), _F32),
        pltpu.VMEM((2, _B, _H), _F32),
        pltpu.VMEM((_B, _H), _F32),
        pltpu.VMEM((_H,), _F32),
        pltpu.VMEM_SHARED((_NP, _H), _F32),
        pltpu.SemaphoreType.DMA,
        pltpu.SemaphoreType.DMA,
        pltpu.SemaphoreType.DMA,
        pltpu.SemaphoreType.DMA,
        pltpu.SemaphoreType.DMA,
        pltpu.SemaphoreType.DMA,
    ]

    def body(tab, w1c2, gs_all, gq_all, dst, ea, zros, out,
             gsv, gqv, dstv, dstsc0, dstsc1, eav, pv, qv, uv, w1cv, s_sh,
             semi0, semi1, semg0, semg1, sems0, sems1):
        ci = lax.axis_index("c")
        s = lax.axis_index("s")
        pltpu.sync_copy(zros.at[pl.ds(s * _ROWS, _ROWS)],
                        s_sh.at[pl.ds(s * _ROWS, _ROWS)])
        pltpu.sync_copy(w1c2.at[ci], w1cv)
        plsc.subcore_barrier()

        base = s * _EPT
        semi = (semi0, semi1)
        semg = (semg0, semg1)
        sems = (sems0, sems1)
        dstsc = (dstsc0, dstsc1)
        w1 = [w1cv[pl.ds(j * 16, 16)] for j in range(_H // 16)]

        def fire_idx(p, k):
            cb = base + k * _B
            pltpu.async_copy(gs_all.at[pl.ds(ci * _E + cb, _B)], gsv.at[p],
                             semi[p])
            pltpu.async_copy(gq_all.at[pl.ds(ci * _E + cb, _B)], gqv.at[p],
                             semi[p])
            pltpu.async_copy(dst.at[pl.ds(cb, _B)], dstv.at[p], semi[p])
            pltpu.async_copy(ea.at[pl.ds(cb, _B)], eav.at[p], semi[p])

        def wait_idx(p):
            pltpu.make_async_copy(gs_all.at[pl.ds(0, _B)], gsv.at[p],
                                  semi[p]).wait()
            pltpu.make_async_copy(gq_all.at[pl.ds(0, _B)], gqv.at[p],
                                  semi[p]).wait()
            pltpu.make_async_copy(dst.at[pl.ds(0, _B)], dstv.at[p],
                                  semi[p]).wait()
            pltpu.make_async_copy(ea.at[pl.ds(0, _B)], eav.at[p],
                                  semi[p]).wait()

        def fire_gather(p):
            pltpu.async_copy(tab.at[gsv.at[p]], pv.at[p], semg[p])
            pltpu.async_copy(tab.at[gqv.at[p]], qv.at[p], semg[p])

        def wait_gather(p):
            pltpu.make_async_copy(tab.at[gsv.at[p]], pv.at[p], semg[p]).wait()
            pltpu.make_async_copy(tab.at[gqv.at[p]], qv.at[p], semg[p]).wait()

        def copy_dstsc(p):
            for o in (0, 16, _B - 16):
                sl = pl.ds(o, 16)
                dstsc[p][sl] = dstv[p, sl]

        def compute(p):
            def edge4(i4, cy):
                for u in range(4):
                    i = i4 * 4 + u
                    eab = eav[p, i, :]
                    for j in range(_H // 16):
                        sl = pl.ds(j * 16, 16)
                        uv[i, sl] = jnp.maximum(
                            pv[p, i, sl] + qv[p, i, sl] + eab * w1[j], 0.0)
                return cy
            lax.fori_loop(0, _B // 4, edge4, 0)

        def fire_scatter(p):
            pltpu.async_copy(uv, s_sh.at[dstsc[p]], sems[p], add=True)

        def wait_scatter(p):
            pltpu.make_async_copy(uv, s_sh.at[dstsc[p]], sems[p]).wait()

        fire_idx(0, 0)
        wait_idx(0)
        fire_gather(0)
        fire_idx(1, 1)

        wait_gather(0)
        copy_dstsc(0)
        wait_idx(1)
        fire_gather(1)
        compute(0)
        fire_scatter(0)
        fire_idx(0, 2)

        wait_gather(1)
        copy_dstsc(1)
        wait_idx(0)
        wait_scatter(0)
        fire_gather(0)
        compute(1)
        fire_scatter(1)
        fire_idx(1, 3)

        def pair(m, cy):
            c0 = 2 * m
            wait_gather(0)
            copy_dstsc(0)
            wait_idx(1)
            wait_scatter(1)
            fire_gather(1)
            compute(0)
            fire_scatter(0)
            fire_idx(0, jnp.minimum(c0 + 2, _NCH - 1))

            wait_gather(1)
            copy_dstsc(1)
            wait_idx(0)
            wait_scatter(0)
            fire_gather(0)
            compute(1)
            fire_scatter(1)
            fire_idx(1, jnp.minimum(c0 + 3, _NCH - 1))
            return cy
        lax.fori_loop(1, _NCH // 2, pair, 0)

        wait_idx(1)
        wait_gather(0)
        wait_scatter(1)

        plsc.subcore_barrier()
        pltpu.sync_copy(s_sh.at[pl.ds(s * _ROWS, _ROWS)],
                        out.at[pl.ds(ci * _NP + s * _ROWS, _ROWS)])

    return pl.kernel(
        body, mesh=mesh,
        out_type=jax.ShapeDtypeStruct((2 * _NP, _H), _F32),
        scratch_types=scratch,
    )


def _sc_deg_fn():
    mesh = plsc.VectorSubcoreMesh(core_axis_name="c", subcore_axis_name="s")
    bd = 80
    ept = _E // (2 * _NSUB)
    nch = ept // bd
    scratch = [
        pltpu.VMEM((bd,), jnp.int32),
        pltpu.VMEM((bd,), jnp.int32),
        pltpu.VMEM((bd, _DEGW), _F32),
        pltpu.VMEM_SHARED((_NP, _DEGW), _F32),
        pltpu.SemaphoreType.DMA,
        pltpu.SemaphoreType.DMA,
        pltpu.SemaphoreType.DMA,
        pltpu.SemaphoreType.DMA,
    ]

    def body(dst, z16, deg_out, dv0, dv1, onesv, deg_sh,
             semi0, semi1, sems0, sems1):
        c = lax.axis_index("c")
        s = lax.axis_index("s")
        pltpu.sync_copy(z16.at[pl.ds(s * _ROWS, _ROWS)],
                        deg_sh.at[pl.ds(s * _ROWS, _ROWS)])

        def _fill(i, carry):
            onesv[i, :] = jnp.ones((_DEGW,), _F32)
            return carry
        lax.fori_loop(0, bd, _fill, 0)
        plsc.subcore_barrier()

        base = (c * _NSUB + s) * ept
        dv = (dv0, dv1)
        semi = (semi0, semi1)
        sems = (sems0, sems1)

        def fire_idx(p, k):
            kc = jnp.minimum(k, nch - 1)
            pltpu.async_copy(dst.at[pl.ds(base + kc * bd, bd)], dv[p], semi[p])

        def wait_idx(p):
            pltpu.make_async_copy(dst.at[pl.ds(0, bd)], dv[p], semi[p]).wait()

        def fire_sc(p):
            pltpu.async_copy(onesv, deg_sh.at[dv[p]], sems[p], add=True)

        def wait_sc(p):
            pltpu.make_async_copy(onesv, deg_sh.at[dv[p]], sems[p]).wait()

        fire_idx(0, 0)
        wait_idx(0)
        fire_idx(1, 1)
        fire_sc(0)

        def pair(m, cy):
            c1 = 2 * m + 1
            wait_idx(1)
            wait_sc(0)
            fire_idx(0, c1 + 1)
            fire_sc(1)

            wait_idx(0)
            wait_sc(1)
            fire_idx(1, c1 + 2)
            fire_sc(0)
            return cy
        lax.fori_loop(0, (nch - 1) // 2, pair, 0)

        wait_idx(1)
        wait_sc(0)

        plsc.subcore_barrier()
        pltpu.sync_copy(deg_sh.at[pl.ds(s * _ROWS, _ROWS)],
                        deg_out.at[c, pl.ds(s * _ROWS, _ROWS)])

    return pl.kernel(
        body, mesh=mesh,
        out_type=jax.ShapeDtypeStruct((2, _NP, _DEGW), _F32),
        scratch_types=scratch,
    )


@functools.cache
def _get_sc_kernels():
    return _sc_edge_fn(), _sc_deg_fn()


def kernel(atom_features, edge_index, edge_attr, enc_W, enc_b, enc_g, enc_bt,
           msg_W1, msg_b1, msg_W2, msg_b2, gru_Wih, gru_bih, gru_Whh, gru_bhh,
           ln_g, ln_b, r_W1, r_b1, r_W2, r_b2, r_W3, r_b3):
    natom = atom_features.shape[1]
    src = edge_index[0]
    dst = edge_index[1]
    ea16 = jnp.broadcast_to(edge_attr, (_E, 16))

    atomp = jnp.zeros((_N, _H), _F32).at[:, :natom].set(atom_features)
    wp = jnp.zeros((_H, _H), _F32).at[:natom, :].set(enc_W)
    x = _enc(atomp, wp, enc_b.reshape(1, _H), enc_g.reshape(1, _H),
             enc_bt.reshape(1, _H))

    zros = jnp.zeros((_NP, _H), _F32)
    z16 = jnp.zeros((_NP, _DEGW), _F32)
    gs_all = jnp.concatenate([src, src + _N])
    gq_all = jnp.concatenate([dst + 2 * _N, dst + 3 * _N])
    sc_edge, sc_deg = _get_sc_kernels()
    degp = sc_deg(dst, z16)
    deg128 = jnp.broadcast_to(degp[0, :_N, :1] + degp[1, :_N, :1], (_N, _H))
    def msg_prep(l):
        W1 = msg_W1[l]
        wst = jnp.stack([W1[:_H], W1[_H:2 * _H]])
        b1 = msg_b1[l]
        bst8 = (jnp.zeros((8, _H), _F32)
                .at[2].set(b1[:_H]).at[3].set(b1[_H:]))
        w1c2 = W1[2 * _H].reshape(2, _H)
        return wst, bst8, w1c2

    hw = _H // 2
    w2p = jnp.zeros((_H, _H), _F32).at[:, :hw].set(r_W2)
    b2p = jnp.zeros((1, _H), _F32).at[0, :hw].set(r_b2)
    w3p = jnp.zeros((_H, _H), _F32).at[:hw, :113].set(r_W3)
    b3p = jnp.zeros((1, _H), _F32).at[0, :113].set(r_b3)

    wst, bst8, w1c2 = msg_prep(0)
    tab = _pq(x, wst, bst8)
    out = None
    for l in range(_NL):
        S2 = sc_edge(tab, w1c2, gs_all, gq_all, dst, ea16, zros)
        W2 = msg_W2[l]
        w2st = jnp.stack([W2[:_H], W2[_H:]])
        gru = (w2st, msg_b2[l].reshape(1, _H),
               gru_Wih[l], gru_bih[l].reshape(1, 3 * _H),
               gru_Whh[l], gru_bhh[l].reshape(1, 3 * _H),
               ln_g[l].reshape(1, _H), ln_b[l].reshape(1, _H))
        if l < _NL - 1:
            x = _tcb(S2, x, deg128, *gru)
            wst, bst8, w1c2 = msg_prep(l + 1)
            tab = _pq(x, wst, bst8)
        else:
            out = _tcb_final(S2, x, deg128, *gru,
                             r_W1, r_b1.reshape(1, _H), w2p, b2p, w3p, b3p)
    return out[:, :113]


# speedup vs baseline: 1.0007x; 1.0007x over previous
import functools

import jax
import jax.numpy as jnp
from jax import lax
from jax.experimental import pallas as pl
from jax.experimental.pallas import tpu as pltpu
from jax.experimental.pallas import tpu_sc as plsc

_N = 10000
_NP = 10240
_E = 320000
_H = 128
_NL = 3
_BS = 400
_NB = _N // _BS
_B = 40
_NSUB = 16
_EPT = _E // _NSUB
_NCH = _EPT // _B
_ROWS = _NP // _NSUB
_DEGW = 16
_F32 = jnp.float32


def _enc_body(a_ref, w_ref, b_ref, g_ref, bt_ref, o_ref):
    x = jnp.dot(a_ref[...], w_ref[...], preferred_element_type=_F32) + b_ref[...]
    x = jnp.maximum(x, 0.0)
    m = jnp.mean(x, axis=-1, keepdims=True)
    v = jnp.mean((x - m) ** 2, axis=-1, keepdims=True)
    o_ref[...] = (x - m) * lax.rsqrt(v + 1e-5) * g_ref[...] + bt_ref[...]


def _enc(atomp, wp, b, g, bt):
    return pl.pallas_call(
        _enc_body,
        grid=(_NB,),
        in_specs=[
            pl.BlockSpec((_BS, _H), lambda i: (i, 0)),
            pl.BlockSpec((_H, _H), lambda i: (0, 0)),
            pl.BlockSpec((1, _H), lambda i: (0, 0)),
            pl.BlockSpec((1, _H), lambda i: (0, 0)),
            pl.BlockSpec((1, _H), lambda i: (0, 0)),
        ],
        out_specs=pl.BlockSpec((_BS, _H), lambda i: (i, 0)),
        out_shape=jax.ShapeDtypeStruct((_N, _H), _F32),
    )(atomp, wp, b, g, bt)


def _pq_body(x_ref, w_ref, b_ref, o_ref):
    r = pl.program_id(0) * 2 + pl.program_id(1)
    rows = lax.broadcasted_iota(jnp.int32, (8, _H), 0)
    b = jnp.sum(jnp.where(rows == r, b_ref[...], 0.0), axis=0, keepdims=True)
    o_ref[...] = (
        jnp.dot(x_ref[...], w_ref[0], preferred_element_type=_F32) + b
    )


def _pq(x, wst, bst8):
    return pl.pallas_call(
        _pq_body,
        grid=(2, 2, _NB),
        in_specs=[
            pl.BlockSpec((_BS, _H), lambda t, c, i: (i, 0)),
            pl.BlockSpec((1, _H, _H), lambda t, c, i: (t, 0, c)),
            pl.BlockSpec((8, _H), lambda t, c, i: (0, 0)),
        ],
        out_specs=pl.BlockSpec((_BS, _H), lambda t, c, i: (t * 2 * _NB + c * _NB + i, 0)),
        out_shape=jax.ShapeDtypeStruct((4 * _N, _H), _F32),
    )(x, wst, bst8)


def _gru_ln(s0, s1, x, deg, w2_ref, b2_ref, wih_ref, bih_ref,
            whh_ref, bhh_ref, lg_ref, lb_ref):
    aggr = (
        jnp.dot(s0, w2_ref[0], preferred_element_type=_F32)
        + jnp.dot(s1, w2_ref[1], preferred_element_type=_F32)
        + deg * b2_ref[...]
    )
    gi = jnp.dot(aggr, wih_ref[...], preferred_element_type=_F32) + bih_ref[...]
    gh = jnp.dot(x, whh_ref[...], preferred_element_type=_F32) + bhh_ref[...]
    r = jax.nn.sigmoid(gi[:, :_H] + gh[:, :_H])
    z = jax.nn.sigmoid(gi[:, _H:2 * _H] + gh[:, _H:2 * _H])
    n = jnp.tanh(gi[:, 2 * _H:] + r * gh[:, 2 * _H:])
    h = x + (1.0 - z) * n + z * x
    m = jnp.mean(h, axis=-1, keepdims=True)
    v = jnp.mean((h - m) ** 2, axis=-1, keepdims=True)
    return (h - m) * lax.rsqrt(v + 1e-5) * lg_ref[...] + lb_ref[...]


def _tcb_body(s0_ref, s1_ref, x_ref, deg_ref, w2_ref, b2_ref, wih_ref,
              bih_ref, whh_ref, bhh_ref, lg_ref, lb_ref, o_ref):
    o_ref[...] = _gru_ln(s0_ref[...], s1_ref[...], x_ref[...], deg_ref[...],
                         w2_ref, b2_ref, wih_ref, bih_ref, whh_ref, bhh_ref,
                         lg_ref, lb_ref)


def _tcb(S2, x, deg128, w2st, b2, wih, bih, whh, bhh, lg, lb):
    return pl.pallas_call(
        _tcb_body,
        grid=(_NB,),
        in_specs=[
            pl.BlockSpec((_BS, _H), lambda i: (i, 0)),
            pl.BlockSpec((_BS, _H), lambda i: (i, 0)),
            pl.BlockSpec((_BS, _H), lambda i: (i, 0)),
            pl.BlockSpec((_BS, _H), lambda i: (i, 0)),
            pl.BlockSpec((2, _H, _H), lambda i: (0, 0, 0)),
            pl.BlockSpec((1, _H), lambda i: (0, 0)),
            pl.BlockSpec((_H, 3 * _H), lambda i: (0, 0)),
            pl.BlockSpec((1, 3 * _H), lambda i: (0, 0)),
            pl.BlockSpec((_H, 3 * _H), lambda i: (0, 0)),
            pl.BlockSpec((1, 3 * _H), lambda i: (0, 0)),
            pl.BlockSpec((1, _H), lambda i: (0, 0)),
            pl.BlockSpec((1, _H), lambda i: (0, 0)),
        ],
        out_specs=pl.BlockSpec((_BS, _H), lambda i: (i, 0)),
        out_shape=jax.ShapeDtypeStruct((_N, _H), _F32),
    )(S2[:_N], S2[_NP:_NP + _N], x, deg128, w2st, b2, wih, bih, whh, bhh,
      lg, lb)


def _tcb_final_body(s0_ref, s1_ref, x_ref, deg_ref, w2_ref, b2_ref, wih_ref,
                    bih_ref, whh_ref, bhh_ref, lg_ref, lb_ref, w1_ref, b1_ref,
                    w2r_ref, b2r_ref, w3_ref, b3_ref, o_ref, sum_acc, max_acc):
    @pl.when(pl.program_id(0) == 0)
    def _init():
        sum_acc[...] = jnp.zeros_like(sum_acc)
        max_acc[...] = jnp.full_like(max_acc, -jnp.inf)

    h = _gru_ln(s0_ref[...], s1_ref[...], x_ref[...], deg_ref[...],
                w2_ref, b2_ref, wih_ref, bih_ref, whh_ref, bhh_ref,
                lg_ref, lb_ref)
    sum_acc[...] += jnp.sum(h, axis=0, keepdims=True)
    max_acc[...] = jnp.maximum(max_acc[...], jnp.max(h, axis=0, keepdims=True))

    @pl.when(pl.program_id(0) == _NB - 1)
    def _final():
        g = jnp.concatenate([sum_acc[...] / float(_N), max_acc[...]], axis=-1)
        h1 = jnp.maximum(
            jnp.dot(g, w1_ref[...], preferred_element_type=_F32) + b1_ref[...], 0.0)
        h2 = jnp.maximum(
            jnp.dot(h1, w2r_ref[...], preferred_element_type=_F32) + b2r_ref[...], 0.0)
        o_ref[...] = jnp.dot(h2, w3_ref[...], preferred_element_type=_F32) + b3_ref[...]


def _tcb_final(S2, x, deg128, w2st, b2, wih, bih, whh, bhh, lg, lb,
               rw1, rb1, rw2, rb2, rw3, rb3):
    return pl.pallas_call(
        _tcb_final_body,
        grid=(_NB,),
        in_specs=[
            pl.BlockSpec((_BS, _H), lambda i: (i, 0)),
            pl.BlockSpec((_BS, _H), lambda i: (i, 0)),
            pl.BlockSpec((_BS, _H), lambda i: (i, 0)),
            pl.BlockSpec((_BS, _H), lambda i: (i, 0)),
            pl.BlockSpec((2, _H, _H), lambda i: (0, 0, 0)),
            pl.BlockSpec((1, _H), lambda i: (0, 0)),
            pl.BlockSpec((_H, 3 * _H), lambda i: (0, 0)),
            pl.BlockSpec((1, 3 * _H), lambda i: (0, 0)),
            pl.BlockSpec((_H, 3 * _H), lambda i: (0, 0)),
            pl.BlockSpec((1, 3 * _H), lambda i: (0, 0)),
            pl.BlockSpec((1, _H), lambda i: (0, 0)),
            pl.BlockSpec((1, _H), lambda i: (0, 0)),
            pl.BlockSpec((2 * _H, _H), lambda i: (0, 0)),
            pl.BlockSpec((1, _H), lambda i: (0, 0)),
            pl.BlockSpec((_H, _H), lambda i: (0, 0)),
            pl.BlockSpec((1, _H), lambda i: (0, 0)),
            pl.BlockSpec((_H, _H), lambda i: (0, 0)),
            pl.BlockSpec((1, _H), lambda i: (0, 0)),
        ],
        out_specs=pl.BlockSpec((1, _H), lambda i: (0, 0)),
        out_shape=jax.ShapeDtypeStruct((1, _H), _F32),
        scratch_shapes=[
            pltpu.VMEM((1, _H), _F32),
            pltpu.VMEM((1, _H), _F32),
        ],
    )(S2[:_N], S2[_NP:_NP + _N], x, deg128, w2st, b2, wih, bih, whh, bhh,
      lg, lb, rw1, rb1, rw2, rb2, rw3, rb3)


def _sc_edge_fn():
    mesh = plsc.VectorSubcoreMesh(core_axis_name="c", subcore_axis_name="s")
    scratch = [
        pltpu.VMEM((2, 2 * _B), jnp.int32),
        pltpu.VMEM((2, _B), jnp.int32),
        pltpu.VMEM((_B,), jnp.int32),
        pltpu.VMEM((_B,), jnp.int32),
        pltpu.VMEM((2, _B, 16), _F32),
        pltpu.VMEM((2, 2 * _B, _H), _F32),
        pltpu.VMEM((_B, _H), _F32),
        pltpu.VMEM((_H,), _F32),
        pltpu.VMEM_SHARED((_NP, _H), _F32),
        pltpu.SemaphoreType.DMA,
        pltpu.SemaphoreType.DMA,
        pltpu.SemaphoreType.DMA,
        pltpu.SemaphoreType.DMA,
        pltpu.SemaphoreType.DMA,
        pltpu.SemaphoreType.DMA,
    ]

    def body(tab, w1c2, gs_all, gq_all, dst, ea, zros, out,
             gidx, dstv, dstsc0, dstsc1, eav, gv, uv, w1cv, s_sh,
             semi0, semi1, semg0, semg1, sems0, sems1):
        ci = lax.axis_index("c")
        s = lax.axis_index("s")
        pltpu.sync_copy(zros.at[pl.ds(s * _ROWS, _ROWS)],
                        s_sh.at[pl.ds(s * _ROWS, _ROWS)])
        pltpu.sync_copy(w1c2.at[ci], w1cv)
        plsc.subcore_barrier()

        base = s * _EPT
        semi = (semi0, semi1)
        semg = (semg0, semg1)
        sems = (sems0, sems1)
        dstsc = (dstsc0, dstsc1)
        w1 = [w1cv[pl.ds(j * 16, 16)] for j in range(_H // 16)]

        def fire_idx(p, k):
            cb = base + k * _B
            pltpu.async_copy(gs_all.at[pl.ds(ci * _E + cb, _B)],
                             gidx.at[p, pl.ds(0, _B)], semi[p])
            pltpu.async_copy(gq_all.at[pl.ds(ci * _E + cb, _B)],
                             gidx.at[p, pl.ds(_B, _B)], semi[p])
            pltpu.async_copy(dst.at[pl.ds(cb, _B)], dstv.at[p], semi[p])
            pltpu.async_copy(ea.at[pl.ds(cb, _B)], eav.at[p], semi[p])

        def wait_idx(p):
            pltpu.make_async_copy(gs_all.at[pl.ds(0, _B)],
                                  gidx.at[p, pl.ds(0, _B)], semi[p]).wait()
            pltpu.make_async_copy(gq_all.at[pl.ds(0, _B)],
                                  gidx.at[p, pl.ds(_B, _B)], semi[p]).wait()
            pltpu.make_async_copy(dst.at[pl.ds(0, _B)], dstv.at[p],
                                  semi[p]).wait()
            pltpu.make_async_copy(ea.at[pl.ds(0, _B)], eav.at[p],
                                  semi[p]).wait()

        def fire_gather(p):
            pltpu.async_copy(tab.at[gidx.at[p]], gv.at[p], semg[p])

        def wait_gather(p):
            pltpu.make_async_copy(tab.at[gidx.at[p]], gv.at[p],
                                  semg[p]).wait()

        def copy_dstsc(p):
            for o in (0, 16, _B - 16):
                sl = pl.ds(o, 16)
                dstsc[p][sl] = dstv[p, sl]

        def compute(p):
            def edge4(i4, cy):
                for u in range(4):
                    i = i4 * 4 + u
                    eab = eav[p, i, :]
                    for j in range(_H // 16):
                        sl = pl.ds(j * 16, 16)
                        uv[i, sl] = jnp.maximum(
                            gv[p, i, sl] + gv[p, _B + i, sl] + eab * w1[j],
                            0.0)
                return cy
            lax.fori_loop(0, _B // 4, edge4, 0)

        def fire_scatter(p):
            pltpu.async_copy(uv, s_sh.at[dstsc[p]], sems[p], add=True)

        def wait_scatter(p):
            pltpu.make_async_copy(uv, s_sh.at[dstsc[p]], sems[p]).wait()

        fire_idx(0, 0)
        wait_idx(0)
        fire_gather(0)
        fire_idx(1, 1)

        wait_gather(0)
        copy_dstsc(0)
        wait_idx(1)
        fire_gather(1)
        compute(0)
        fire_scatter(0)
        fire_idx(0, 2)

        wait_gather(1)
        copy_dstsc(1)
        wait_idx(0)
        wait_scatter(0)
        fire_gather(0)
        compute(1)
        fire_scatter(1)
        fire_idx(1, 3)

        def pair(m, cy):
            c0 = 2 * m
            wait_gather(0)
            copy_dstsc(0)
            wait_idx(1)
            wait_scatter(1)
            fire_gather(1)
            compute(0)
            fire_scatter(0)
            fire_idx(0, jnp.minimum(c0 + 2, _NCH - 1))

            wait_gather(1)
            copy_dstsc(1)
            wait_idx(0)
            wait_scatter(0)
            fire_gather(0)
            compute(1)
            fire_scatter(1)
            fire_idx(1, jnp.minimum(c0 + 3, _NCH - 1))
            return cy
        lax.fori_loop(1, _NCH // 2, pair, 0)

        wait_idx(1)
        wait_gather(0)
        wait_scatter(1)

        plsc.subcore_barrier()
        pltpu.sync_copy(s_sh.at[pl.ds(s * _ROWS, _ROWS)],
                        out.at[pl.ds(ci * _NP + s * _ROWS, _ROWS)])

    return pl.kernel(
        body, mesh=mesh,
        out_type=jax.ShapeDtypeStruct((2 * _NP, _H), _F32),
        scratch_types=scratch,
    )


def _sc_deg_fn():
    mesh = plsc.VectorSubcoreMesh(core_axis_name="c", subcore_axis_name="s")
    bd = 80
    ept = _E // (2 * _NSUB)
    nch = ept // bd
    scratch = [
        pltpu.VMEM((bd,), jnp.int32),
        pltpu.VMEM((bd,), jnp.int32),
        pltpu.VMEM((bd, _DEGW), _F32),
        pltpu.VMEM_SHARED((_NP, _DEGW), _F32),
        pltpu.SemaphoreType.DMA,
        pltpu.SemaphoreType.DMA,
        pltpu.SemaphoreType.DMA,
        pltpu.SemaphoreType.DMA,
    ]

    def body(dst, z16, deg_out, dv0, dv1, onesv, deg_sh,
             semi0, semi1, sems0, sems1):
        c = lax.axis_index("c")
        s = lax.axis_index("s")
        pltpu.sync_copy(z16.at[pl.ds(s * _ROWS, _ROWS)],
                        deg_sh.at[pl.ds(s * _ROWS, _ROWS)])

        def _fill(i, carry):
            onesv[i, :] = jnp.ones((_DEGW,), _F32)
            return carry
        lax.fori_loop(0, bd, _fill, 0)
        plsc.subcore_barrier()

        base = (c * _NSUB + s) * ept
        dv = (dv0, dv1)
        semi = (semi0, semi1)
        sems = (sems0, sems1)

        def fire_idx(p, k):
            kc = jnp.minimum(k, nch - 1)
            pltpu.async_copy(dst.at[pl.ds(base + kc * bd, bd)], dv[p], semi[p])

        def wait_idx(p):
            pltpu.make_async_copy(dst.at[pl.ds(0, bd)], dv[p], semi[p]).wait()

        def fire_sc(p):
            pltpu.async_copy(onesv, deg_sh.at[dv[p]], sems[p], add=True)

        def wait_sc(p):
            pltpu.make_async_copy(onesv, deg_sh.at[dv[p]], sems[p]).wait()

        fire_idx(0, 0)
        wait_idx(0)
        fire_idx(1, 1)
        fire_sc(0)

        def pair(m, cy):
            c1 = 2 * m + 1
            wait_idx(1)
            wait_sc(0)
            fire_idx(0, c1 + 1)
            fire_sc(1)

            wait_idx(0)
            wait_sc(1)
            fire_idx(1, c1 + 2)
            fire_sc(0)
            return cy
        lax.fori_loop(0, (nch - 1) // 2, pair, 0)

        wait_idx(1)
        wait_sc(0)

        plsc.subcore_barrier()
        pltpu.sync_copy(deg_sh.at[pl.ds(s * _ROWS, _ROWS)],
                        deg_out.at[c, pl.ds(s * _ROWS, _ROWS)])

    return pl.kernel(
        body, mesh=mesh,
        out_type=jax.ShapeDtypeStruct((2, _NP, _DEGW), _F32),
        scratch_types=scratch,
    )


@functools.cache
def _get_sc_kernels():
    return _sc_edge_fn(), _sc_deg_fn()


def kernel(atom_features, edge_index, edge_attr, enc_W, enc_b, enc_g, enc_bt,
           msg_W1, msg_b1, msg_W2, msg_b2, gru_Wih, gru_bih, gru_Whh, gru_bhh,
           ln_g, ln_b, r_W1, r_b1, r_W2, r_b2, r_W3, r_b3):
    natom = atom_features.shape[1]
    src = edge_index[0]
    dst = edge_index[1]
    ea16 = jnp.broadcast_to(edge_attr, (_E, 16))

    atomp = jnp.zeros((_N, _H), _F32).at[:, :natom].set(atom_features)
    wp = jnp.zeros((_H, _H), _F32).at[:natom, :].set(enc_W)
    x = _enc(atomp, wp, enc_b.reshape(1, _H), enc_g.reshape(1, _H),
             enc_bt.reshape(1, _H))

    zros = jnp.zeros((_NP, _H), _F32)
    z16 = jnp.zeros((_NP, _DEGW), _F32)
    gs_all = jnp.concatenate([src, src + _N])
    gq_all = jnp.concatenate([dst + 2 * _N, dst + 3 * _N])
    sc_edge, sc_deg = _get_sc_kernels()
    degp = sc_deg(dst, z16)
    deg128 = jnp.broadcast_to(degp[0, :_N, :1] + degp[1, :_N, :1], (_N, _H))
    def msg_prep(l):
        W1 = msg_W1[l]
        wst = jnp.stack([W1[:_H], W1[_H:2 * _H]])
        b1 = msg_b1[l]
        bst8 = (jnp.zeros((8, _H), _F32)
                .at[2].set(b1[:_H]).at[3].set(b1[_H:]))
        w1c2 = W1[2 * _H].reshape(2, _H)
        return wst, bst8, w1c2

    hw = _H // 2
    w2p = jnp.zeros((_H, _H), _F32).at[:, :hw].set(r_W2)
    b2p = jnp.zeros((1, _H), _F32).at[0, :hw].set(r_b2)
    w3p = jnp.zeros((_H, _H), _F32).at[:hw, :113].set(r_W3)
    b3p = jnp.zeros((1, _H), _F32).at[0, :113].set(r_b3)

    wst, bst8, w1c2 = msg_prep(0)
    tab = _pq(x, wst, bst8)
    out = None
    for l in range(_NL):
        S2 = sc_edge(tab, w1c2, gs_all, gq_all, dst, ea16, zros)
        W2 = msg_W2[l]
        w2st = jnp.stack([W2[:_H], W2[_H:]])
        gru = (w2st, msg_b2[l].reshape(1, _H),
               gru_Wih[l], gru_bih[l].reshape(1, 3 * _H),
               gru_Whh[l], gru_bhh[l].reshape(1, 3 * _H),
               ln_g[l].reshape(1, _H), ln_b[l].reshape(1, _H))
        if l < _NL - 1:
            x = _tcb(S2, x, deg128, *gru)
            wst, bst8, w1c2 = msg_prep(l + 1)
            tab = _pq(x, wst, bst8)
        else:
            out = _tcb_final(S2, x, deg128, *gru,
                             r_W1, r_b1.reshape(1, _H), w2p, b2p, w3p, b3p)
    return out[:, :113]
